# Initial kernel scaffold; baseline (speedup 1.0000x reference)
import functools

import jax
import jax.numpy as jnp
from jax import lax
from jax.experimental import pallas as pl
from jax.experimental.pallas import tpu as pltpu
from jax.experimental.pallas import tpu_sc as plsc

_EMB = 32
_NC, _NS = 2, 16
_NW = _NC * _NS
_IDX_ROW = 128
_CHUNK_ROWS = 8
_CHUNK = _CHUNK_ROWS * _IDX_ROW


def _sc_gather(idx2d, table):
    total = idx2d.shape[0] * idx2d.shape[1]
    per_w = total // _NW
    n_chunks = per_w // _CHUNK
    mesh = plsc.VectorSubcoreMesh(
        core_axis_name="c", subcore_axis_name="s",
        num_cores=_NC, num_subcores=_NS)

    @functools.partial(
        pl.kernel,
        out_type=jax.ShapeDtypeStruct((total, _EMB), jnp.float32),
        mesh=mesh,
        scratch_types=[
            pltpu.VMEM((_CHUNK_ROWS, _IDX_ROW), jnp.int32),
            pltpu.VMEM((_CHUNK, _EMB), jnp.float32),
            pltpu.SemaphoreType.DMA,
        ],
    )
    def k(idx_hbm, table_hbm, out_hbm, idx_v, rows_v, sem):
        wid = lax.axis_index("s") * _NC + lax.axis_index("c")
        row0 = wid * (per_w // _IDX_ROW)

        @pl.loop(0, n_chunks)
        def chunk(g):
            r = row0 + g * _CHUNK_ROWS
            pltpu.sync_copy(idx_hbm.at[pl.ds(r, _CHUNK_ROWS)], idx_v)
            copies = [
                pltpu.async_copy(
                    table_hbm.at[idx_v.at[j]],
                    rows_v.at[pl.ds(j * _IDX_ROW, _IDX_ROW)],
                    sem)
                for j in range(_CHUNK_ROWS)
            ]
            for c in copies:
                c.wait()
            pltpu.sync_copy(rows_v, out_hbm.at[pl.ds(r * _IDX_ROW, _CHUNK)])

    return k(idx2d, table)


def kernel(x, W):
    b, h = x.shape
    idx2d = x.astype(jnp.int32).reshape((b * h) // _IDX_ROW, _IDX_ROW)
    out = _sc_gather(idx2d, W)
    return out.reshape(b, h, _EMB)


# speedup vs baseline: 5.0417x; 5.0417x over previous
import functools

import jax
import jax.numpy as jnp
from jax import lax
from jax.experimental import pallas as pl
from jax.experimental.pallas import tpu as pltpu
from jax.experimental.pallas import tpu_sc as plsc

_EMB = 32
_NC, _NS = 2, 16
_NW = _NC * _NS
_IDX_ROW = 128
_CHUNK_ROWS = 8
_CHUNK = _CHUNK_ROWS * _IDX_ROW


def _sc_gather(idx2d, table, total):
    per_w = total // _NW
    n = per_w // _CHUNK
    rows_per_w = per_w // _IDX_ROW
    mesh = plsc.VectorSubcoreMesh(
        core_axis_name="c", subcore_axis_name="s",
        num_cores=_NC, num_subcores=_NS)

    @functools.partial(
        pl.kernel,
        out_type=jax.ShapeDtypeStruct((total, _EMB), jnp.float32),
        mesh=mesh,
        scratch_types=[
            pltpu.VMEM((2, _CHUNK_ROWS, _IDX_ROW), jnp.int32),
            pltpu.VMEM((2, _CHUNK, _EMB), jnp.float32),
            pltpu.SemaphoreType.DMA,
            pltpu.SemaphoreType.DMA,
            pltpu.SemaphoreType.DMA,
            pltpu.SemaphoreType.DMA,
            pltpu.SemaphoreType.DMA,
            pltpu.SemaphoreType.DMA,
        ],
        compiler_params=pltpu.CompilerParams(use_tc_tiling_on_sc=False),
    )
    def k(idx_hbm, table_hbm, out_hbm, idx_v, rows_v,
          is0, is1, gs0, gs1, ss0, ss1):
        idx_sem = (is0, is1)
        gat_sem = (gs0, gs1)
        st_sem = (ss0, ss1)
        wid = lax.axis_index("s") * _NC + lax.axis_index("c")
        row0 = wid * rows_per_w

        def idx_desc(g, b):
            return pltpu.make_async_copy(
                idx_hbm.at[pl.ds(row0 + g * _CHUNK_ROWS, _CHUNK_ROWS)],
                idx_v.at[b], idx_sem[b])

        def gat_descs(b):
            return [
                pltpu.make_async_copy(
                    table_hbm.at[idx_v.at[b].at[j]],
                    rows_v.at[b].at[pl.ds(j * _IDX_ROW, _IDX_ROW)],
                    gat_sem[b])
                for j in range(_CHUNK_ROWS)
            ]

        def st_desc(g, b):
            return pltpu.make_async_copy(
                rows_v.at[b],
                out_hbm.at[pl.ds((row0 + g * _CHUNK_ROWS) * _IDX_ROW, _CHUNK)],
                st_sem[b])

        def gat_start(b):
            for d in gat_descs(b):
                d.start()

        def gat_wait(b):
            for d in gat_descs(b):
                d.wait()

        idx_desc(0, 0).start()
        idx_desc(0, 0).wait()
        gat_start(0)
        idx_desc(1, 1).start()
        idx_desc(1, 1).wait()
        gat_start(1)
        gat_wait(0)
        idx_desc(2, 0).start()
        st_desc(0, 0).start()

        @pl.loop(0, (n - 2) // 2)
        def main(t):
            for b in (0, 1):
                g = 2 + 2 * t + b
                st_desc(g - 2, b).wait()
                idx_desc(g, b).wait()
                gat_start(b)
                gat_wait(1 - b)
                idx_desc(g + 1, 1 - b).start()
                st_desc(g - 1, 1 - b).start()

        gat_wait(1)
        st_desc(n - 1, 1).start()
        st_desc(n - 2, 0).wait()
        st_desc(n - 1, 1).wait()
        idx_desc(n, 0).wait()

    return k(idx2d, table)


def kernel(x, W):
    b, h = x.shape
    total = b * h
    idx2d = x.astype(jnp.int32).reshape(total // _IDX_ROW, _IDX_ROW)
    idx2d = jnp.pad(idx2d, ((0, _CHUNK_ROWS), (0, 0)))
    out = _sc_gather(idx2d, W, total)
    return out.reshape(b, h, _EMB)


# speedup vs baseline: 5.0437x; 1.0004x over previous
import functools

import jax
import jax.numpy as jnp
from jax import lax
from jax.experimental import pallas as pl
from jax.experimental.pallas import tpu as pltpu
from jax.experimental.pallas import tpu_sc as plsc

_EMB = 32
_NC, _NS = 2, 16
_NW = _NC * _NS
_CHUNK = 1024


def _sc_gather(idx_flat, table, total):
    per_w = total // _NW
    n = per_w // _CHUNK
    mesh = plsc.VectorSubcoreMesh(
        core_axis_name="c", subcore_axis_name="s",
        num_cores=_NC, num_subcores=_NS)

    @functools.partial(
        pl.kernel,
        out_type=jax.ShapeDtypeStruct((total, _EMB), jnp.float32),
        mesh=mesh,
        scratch_types=[
            pltpu.VMEM((2, _CHUNK), jnp.int32),
            pltpu.VMEM((2, _CHUNK, _EMB), jnp.float32),
            pltpu.SemaphoreType.DMA,
            pltpu.SemaphoreType.DMA,
            pltpu.SemaphoreType.DMA,
            pltpu.SemaphoreType.DMA,
            pltpu.SemaphoreType.DMA,
            pltpu.SemaphoreType.DMA,
        ],
        compiler_params=pltpu.CompilerParams(use_tc_tiling_on_sc=False),
    )
    def k(idx_hbm, table_hbm, out_hbm, idx_v, rows_v,
          is0, is1, gs0, gs1, ss0, ss1):
        idx_sem = (is0, is1)
        gat_sem = (gs0, gs1)
        st_sem = (ss0, ss1)
        wid = lax.axis_index("s") * _NC + lax.axis_index("c")
        base = wid * per_w

        def idx_desc(g, b):
            return pltpu.make_async_copy(
                idx_hbm.at[pl.ds(base + g * _CHUNK, _CHUNK)],
                idx_v.at[b], idx_sem[b])

        def gat_desc(b):
            return pltpu.make_async_copy(
                table_hbm.at[idx_v.at[b]], rows_v.at[b], gat_sem[b])

        def st_desc(g, b):
            return pltpu.make_async_copy(
                rows_v.at[b],
                out_hbm.at[pl.ds(base + g * _CHUNK, _CHUNK)],
                st_sem[b])

        idx_desc(0, 0).start()
        idx_desc(0, 0).wait()
        gat_desc(0).start()
        idx_desc(1, 1).start()
        idx_desc(1, 1).wait()
        gat_desc(1).start()
        gat_desc(0).wait()
        idx_desc(2, 0).start()
        st_desc(0, 0).start()

        @pl.loop(0, (n - 2) // 2)
        def main(t):
            for b in (0, 1):
                g = 2 + 2 * t + b
                st_desc(g - 2, b).wait()
                idx_desc(g, b).wait()
                gat_desc(b).start()
                gat_desc(1 - b).wait()
                idx_desc(g + 1, 1 - b).start()
                st_desc(g - 1, 1 - b).start()

        gat_desc(1).wait()
        st_desc(n - 1, 1).start()
        st_desc(n - 2, 0).wait()
        st_desc(n - 1, 1).wait()
        idx_desc(n, 0).wait()

    return k(idx_flat, table)


def kernel(x, W):
    b, h = x.shape
    total = b * h
    idx_flat = x.astype(jnp.int32).reshape(total)
    idx_flat = jnp.pad(idx_flat, (0, _CHUNK))
    out = _sc_gather(idx_flat, W, total)
    return out.reshape(b, h, _EMB)
